# baseline (speedup 1.0000x reference)
import functools

import jax
import jax.numpy as jnp
from jax.experimental import pallas as pl
from jax.experimental.pallas import tpu as pltpu

_VMEM_LIMIT = 48 * 1024 * 1024


def _conv_kernel(xc_ref, xt_ref, xb_ref, w_ref, b_ref, o_ref, *,
                 th, wp, wt, cin, cout, pool, wpo, packed_dx):
    i = pl.program_id(1)
    nh = pl.num_programs(1)

    x_c = xc_ref[0]
    x_t = xt_ref[0]
    x_b = xb_ref[0]
    zrow = jnp.zeros_like(x_t)
    x_t = jnp.where(i == 0, zrow, x_t)
    x_b = jnp.where(i == nh - 1, zrow, x_b)
    strip = jnp.concatenate([x_t, x_c, x_b], axis=0)

    if packed_dx:
        packed = strip
        kc = cin
    else:
        zcol = jnp.zeros((th + 2, 1, cin), strip.dtype)
        left = jnp.concatenate([zcol, strip[:, :wp - 1, :]], axis=1)
        right = jnp.concatenate([strip[:, 1:, :], zcol], axis=1)
        packed = jnp.concatenate([left, strip, right], axis=2)
        kc = 3 * cin

    m = th * wp
    y = None
    for dy in range(3):
        d = jnp.dot(packed[dy:dy + th].reshape(m, kc), w_ref[dy],
                    preferred_element_type=jnp.float32)
        y = d if y is None else y + d
    y = jnp.maximum(y + b_ref[...], 0.0)

    if pool:
        z = y.reshape(th // 2, 2, wp, cout)
        z = jnp.max(z, axis=1)
        z = z.reshape(th // 2, wp // 2, 2, cout)
        y = jnp.max(z, axis=2)
        ho, wo, wto = th // 2, wp // 2, wt // 2
    else:
        y = y.reshape(th, wp, cout)
        ho, wo, wto = th, wp, wt
    if wto < wo:
        col = jax.lax.broadcasted_iota(jnp.int32, (ho, wo, cout), 1)
        y = jnp.where(col < wto, y, 0.0)
    if wpo > wo:
        y = jnp.concatenate(
            [y, jnp.zeros((ho, wpo - wo, cout), y.dtype)], axis=1)
    o_ref[...] = y.reshape(1, ho, wpo, cout).astype(o_ref.dtype)


def _conv(x, w, b, *, th, pool, wt, wpo, packed_dx=False):
    n, h, wp, cin = x.shape
    cout = w.shape[-1]
    if packed_dx:
        wk = w
        kc = w.shape[1]
    else:
        wk = w.reshape(3, 3 * cin, cout)
        kc = 3 * cin
    ho = h // 2 if pool else h
    tho = th // 2 if pool else th
    kfn = functools.partial(_conv_kernel, th=th, wp=wp, wt=wt, cin=cin,
                            cout=cout, pool=pool, wpo=wpo, packed_dx=packed_dx)
    return pl.pallas_call(
        kfn,
        out_shape=jax.ShapeDtypeStruct((n, ho, wpo, cout), x.dtype),
        grid=(n, h // th),
        in_specs=[
            pl.BlockSpec((1, th, wp, cin), lambda n_, i: (n_, i, 0, 0)),
            pl.BlockSpec((1, 1, wp, cin),
                         lambda n_, i: (n_, jnp.maximum(i * th - 1, 0), 0, 0)),
            pl.BlockSpec((1, 1, wp, cin),
                         lambda n_, i: (n_, jnp.minimum(i * th + th, h - 1),
                                        0, 0)),
            pl.BlockSpec((3, kc, cout), lambda n_, i: (0, 0, 0)),
            pl.BlockSpec((1, cout), lambda n_, i: (0, 0)),
        ],
        out_specs=pl.BlockSpec((1, tho, wpo, cout), lambda n_, i: (n_, i, 0, 0)),
        compiler_params=pltpu.CompilerParams(
            dimension_semantics=("parallel", "parallel"),
            vmem_limit_bytes=_VMEM_LIMIT),
    )(x, x, x, wk, b.reshape(1, cout))


def _conv_pair_kernel(xc_ref, xt_ref, xb_ref, w_ref, b_ref, o_ref, *,
                      th, w2, cin, cout, pool):
    i = pl.program_id(1)
    nh = pl.num_programs(1)
    x_c = xc_ref[0]
    x_t = xt_ref[0]
    x_b = xb_ref[0]
    zrow = jnp.zeros_like(x_t)
    x_t = jnp.where(i == 0, zrow, x_t)
    x_b = jnp.where(i == nh - 1, zrow, x_b)
    strip = jnp.concatenate([x_t, x_c, x_b], axis=0)

    zcol = jnp.zeros((th + 2, 1, cin), strip.dtype)
    t0 = jnp.concatenate([zcol, strip[:, :w2 - 1, cin:]], axis=1)
    t3 = jnp.concatenate([strip[:, 1:, :cin], zcol], axis=1)
    packed = jnp.concatenate([t0, strip, t3], axis=2)

    m = th * w2
    k3 = 3 * cin
    y0 = y1 = None
    for dy in range(3):
        seg = packed[dy:dy + th]
        d0 = jnp.dot(seg[:, :, :k3].reshape(m, k3), w_ref[dy],
                     preferred_element_type=jnp.float32)
        d1 = jnp.dot(seg[:, :, cin:].reshape(m, k3), w_ref[dy],
                     preferred_element_type=jnp.float32)
        y0 = d0 if y0 is None else y0 + d0
        y1 = d1 if y1 is None else y1 + d1
    y0 = jnp.maximum(y0 + b_ref[...], 0.0)
    y1 = jnp.maximum(y1 + b_ref[...], 0.0)

    if pool:
        y = jnp.maximum(y0, y1)
        y = y.reshape(th // 2, 2, w2, cout)
        y = jnp.max(y, axis=1)
        o_ref[...] = y.reshape(1, th // 2, w2, cout).astype(o_ref.dtype)
    else:
        y = jnp.concatenate([y0, y1], axis=1)
        o_ref[...] = y.reshape(1, th, w2, 2 * cout).astype(o_ref.dtype)


def _conv_pair(x, w, b, *, th, pool):
    n, h, w2, cin2 = x.shape
    cin = cin2 // 2
    cout = w.shape[-1]
    wk = w.reshape(3, 3 * cin, cout)
    ho = h // 2 if pool else h
    tho = th // 2 if pool else th
    co = cout if pool else 2 * cout
    kfn = functools.partial(_conv_pair_kernel, th=th, w2=w2, cin=cin,
                            cout=cout, pool=pool)
    return pl.pallas_call(
        kfn,
        out_shape=jax.ShapeDtypeStruct((n, ho, w2, co), x.dtype),
        grid=(n, h // th),
        in_specs=[
            pl.BlockSpec((1, th, w2, cin2), lambda n_, i: (n_, i, 0, 0)),
            pl.BlockSpec((1, 1, w2, cin2),
                         lambda n_, i: (n_, jnp.maximum(i * th - 1, 0), 0, 0)),
            pl.BlockSpec((1, 1, w2, cin2),
                         lambda n_, i: (n_, jnp.minimum(i * th + th, h - 1),
                                        0, 0)),
            pl.BlockSpec((3, 3 * cin, cout), lambda n_, i: (0, 0, 0)),
            pl.BlockSpec((1, cout), lambda n_, i: (0, 0)),
        ],
        out_specs=pl.BlockSpec((1, tho, w2, co), lambda n_, i: (n_, i, 0, 0)),
        compiler_params=pltpu.CompilerParams(
            dimension_semantics=("parallel", "parallel"),
            vmem_limit_bytes=_VMEM_LIMIT),
    )(x, x, x, wk, b.reshape(1, cout))


def _stem_kernel(x_ref, w_ref, b_ref, o_ref):
    y = jnp.dot(x_ref[...], w_ref[...], preferred_element_type=jnp.float32)
    o_ref[...] = jnp.maximum(y + b_ref[...], 0.0).astype(o_ref.dtype)


def _stem(x, w, b):
    n, h, ww, cin = x.shape
    cout = w.shape[-1]
    xp = jnp.pad(x, ((0, 0), (1, 1), (1, 1), (0, 0)))
    taps = [xp[:, dy:dy + h, dx:dx + ww, :]
            for dy in range(3) for dx in range(3)]
    xi = jnp.concatenate(taps, axis=-1).reshape(n * h * ww, 9 * cin)
    wk = w.reshape(9 * cin, cout)
    m = n * h * ww
    tm = 4096
    out = pl.pallas_call(
        _stem_kernel,
        out_shape=jax.ShapeDtypeStruct((m, cout), x.dtype),
        grid=(m // tm,),
        in_specs=[
            pl.BlockSpec((tm, 9 * cin), lambda i: (i, 0)),
            pl.BlockSpec((9 * cin, cout), lambda i: (0, 0)),
            pl.BlockSpec((1, cout), lambda i: (0, 0)),
        ],
        out_specs=pl.BlockSpec((tm, cout), lambda i: (i, 0)),
        compiler_params=pltpu.CompilerParams(
            dimension_semantics=("parallel",),
            vmem_limit_bytes=_VMEM_LIMIT),
    )(xi, wk, b.reshape(1, cout))
    return out.reshape(n, h, ww, cout)


def _fc_int8_kernel(x_ref, wq_ref, s_ref, b_ref, o_ref, acc_ref, *, relu):
    k = pl.program_id(1)

    @pl.when(k == 0)
    def _():
        acc_ref[...] = jnp.zeros_like(acc_ref)

    w = wq_ref[...].astype(jnp.bfloat16)
    acc_ref[...] += jnp.dot(x_ref[...], w, preferred_element_type=jnp.float32)

    @pl.when(k == pl.num_programs(1) - 1)
    def _():
        y = acc_ref[...] * s_ref[...] + b_ref[...]
        if relu:
            y = jnp.maximum(y, 0.0)
        o_ref[...] = y.astype(o_ref.dtype)


def _fc_int8(x, wq, s, b, *, relu, tk, tn):
    bsz, kdim = x.shape
    ndim = wq.shape[1]
    kfn = functools.partial(_fc_int8_kernel, relu=relu)
    return pl.pallas_call(
        kfn,
        out_shape=jax.ShapeDtypeStruct((bsz, ndim), x.dtype),
        grid_spec=pltpu.PrefetchScalarGridSpec(
            num_scalar_prefetch=0,
            grid=(ndim // tn, kdim // tk),
            in_specs=[
                pl.BlockSpec((bsz, tk), lambda j, k: (0, k)),
                pl.BlockSpec((tk, tn), lambda j, k: (k, j)),
                pl.BlockSpec((1, tn), lambda j, k: (0, j)),
                pl.BlockSpec((1, tn), lambda j, k: (0, j)),
            ],
            out_specs=pl.BlockSpec((bsz, tn), lambda j, k: (0, j)),
            scratch_shapes=[pltpu.VMEM((bsz, tn), jnp.float32)],
        ),
        compiler_params=pltpu.CompilerParams(
            dimension_semantics=("parallel", "arbitrary"),
            vmem_limit_bytes=_VMEM_LIMIT),
    )(x, wq, s.reshape(1, ndim), b.reshape(1, ndim))


def _fc_kernel(x_ref, w_ref, b_ref, o_ref):
    y = jnp.dot(x_ref[...], w_ref[...], preferred_element_type=jnp.float32)
    o_ref[...] = y + b_ref[...]


def _fc_small(x, w, b):
    bsz = x.shape[0]
    ndim = w.shape[1]
    return pl.pallas_call(
        _fc_kernel,
        out_shape=jax.ShapeDtypeStruct((bsz, ndim), jnp.float32),
        compiler_params=pltpu.CompilerParams(
            vmem_limit_bytes=_VMEM_LIMIT),
    )(x, w, b.reshape(1, ndim))


_PLAN = [
    (4, 28, False, 56, 56),
    (5, 28, False, 56, 56),
    (6, 28, True, 56, 28),
    (7, 28, False, 28, 28),
    (8, 28, False, 28, 28),
    (9, 28, True, 28, 14),
    (10, 14, False, 14, 14),
    (11, 14, False, 14, 14),
    (12, 14, True, 14, 7),
]


def kernel(x,
           conv0_w, conv0_b, conv1_w, conv1_b, conv2_w, conv2_b,
           conv3_w, conv3_b, conv4_w, conv4_b, conv5_w, conv5_b,
           conv6_w, conv6_b, conv7_w, conv7_b, conv8_w, conv8_b,
           conv9_w, conv9_b, conv10_w, conv10_b, conv11_w, conv11_b,
           conv12_w, conv12_b,
           fc0_wq, fc0_s, fc0_b, fc1_wq, fc1_s, fc1_b, fc2_w, fc2_b):
    convs = [(conv0_w, conv0_b), (conv1_w, conv1_b), (conv2_w, conv2_b),
             (conv3_w, conv3_b), (conv4_w, conv4_b), (conv5_w, conv5_b),
             (conv6_w, conv6_b), (conv7_w, conv7_b), (conv8_w, conv8_b),
             (conv9_w, conv9_b), (conv10_w, conv10_b), (conv11_w, conv11_b),
             (conv12_w, conv12_b)]

    h = jnp.transpose(x, (0, 2, 3, 1)).astype(jnp.bfloat16)
    h = _stem(h, convs[0][0], convs[0][1])
    h = h.reshape(16, 224, 112, 128)
    h = _conv_pair(h, convs[1][0], convs[1][1], th=28, pool=True)
    h = _conv(h, convs[2][0], convs[2][1], th=28, pool=False,
              wt=112, wpo=112)
    h = h.reshape(16, 112, 56, 256)
    h = _conv_pair(h, convs[3][0], convs[3][1], th=28, pool=True)
    for li, th, pool, wt, wpo in _PLAN:
        w, b = convs[li]
        h = _conv(h, w, b, th=th, pool=pool, wt=wt, wpo=wpo)

    h = h[:, :, :7, :]
    h = jnp.transpose(h, (0, 3, 1, 2)).reshape(h.shape[0], -1)
    h = _fc_int8(h, fc0_wq, fc0_s, fc0_b, relu=True, tk=1792, tn=2048)
    h = _fc_int8(h, fc1_wq, fc1_s, fc1_b, relu=True, tk=2048, tn=2048)
    return _fc_small(h, fc2_w, fc2_b)


# speedup vs baseline: 1.1883x; 1.1883x over previous
import functools

import jax
import jax.numpy as jnp
from jax.experimental import pallas as pl
from jax.experimental.pallas import tpu as pltpu

_VMEM_LIMIT = 48 * 1024 * 1024


def _conv_kernel(xc_ref, xt_ref, xb_ref, w_ref, b_ref, o_ref, *,
                 th, wp, wt, cin, cout, pool, wpo, packed_dx):
    i = pl.program_id(1)
    nh = pl.num_programs(1)

    x_c = xc_ref[0]
    x_t = xt_ref[0]
    x_b = xb_ref[0]
    zrow = jnp.zeros_like(x_t)
    x_t = jnp.where(i == 0, zrow, x_t)
    x_b = jnp.where(i == nh - 1, zrow, x_b)
    strip = jnp.concatenate([x_t, x_c, x_b], axis=0)

    if packed_dx:
        packed = strip
        kc = cin
    else:
        zcol = jnp.zeros((th + 2, 1, cin), strip.dtype)
        left = jnp.concatenate([zcol, strip[:, :wp - 1, :]], axis=1)
        right = jnp.concatenate([strip[:, 1:, :], zcol], axis=1)
        packed = jnp.concatenate([left, strip, right], axis=2)
        kc = 3 * cin

    m = th * wp
    y = None
    for dy in range(3):
        d = jnp.dot(packed[dy:dy + th].reshape(m, kc), w_ref[dy],
                    preferred_element_type=jnp.float32)
        y = d if y is None else y + d
    y = jnp.maximum(y + b_ref[...], 0.0)

    if pool:
        z = y.reshape(th // 2, 2, wp, cout)
        z = jnp.max(z, axis=1)
        z = z.reshape(th // 2, wp // 2, 2, cout)
        y = jnp.max(z, axis=2)
        ho, wo, wto = th // 2, wp // 2, wt // 2
    else:
        y = y.reshape(th, wp, cout)
        ho, wo, wto = th, wp, wt
    if wto < wo:
        col = jax.lax.broadcasted_iota(jnp.int32, (ho, wo, cout), 1)
        y = jnp.where(col < wto, y, 0.0)
    if wpo > wo:
        y = jnp.concatenate(
            [y, jnp.zeros((ho, wpo - wo, cout), y.dtype)], axis=1)
    o_ref[...] = y.reshape(1, ho, wpo, cout).astype(o_ref.dtype)


def _conv(x, w, b, *, th, pool, wt, wpo, packed_dx=False):
    n, h, wp, cin = x.shape
    cout = w.shape[-1]
    if packed_dx:
        wk = w
        kc = w.shape[1]
    else:
        wk = w.reshape(3, 3 * cin, cout)
        kc = 3 * cin
    ho = h // 2 if pool else h
    tho = th // 2 if pool else th
    kfn = functools.partial(_conv_kernel, th=th, wp=wp, wt=wt, cin=cin,
                            cout=cout, pool=pool, wpo=wpo, packed_dx=packed_dx)
    return pl.pallas_call(
        kfn,
        out_shape=jax.ShapeDtypeStruct((n, ho, wpo, cout), x.dtype),
        grid=(n, h // th),
        in_specs=[
            pl.BlockSpec((1, th, wp, cin), lambda n_, i: (n_, i, 0, 0)),
            pl.BlockSpec((1, 1, wp, cin),
                         lambda n_, i: (n_, jnp.maximum(i * th - 1, 0), 0, 0)),
            pl.BlockSpec((1, 1, wp, cin),
                         lambda n_, i: (n_, jnp.minimum(i * th + th, h - 1),
                                        0, 0)),
            pl.BlockSpec((3, kc, cout), lambda n_, i: (0, 0, 0)),
            pl.BlockSpec((1, cout), lambda n_, i: (0, 0)),
        ],
        out_specs=pl.BlockSpec((1, tho, wpo, cout), lambda n_, i: (n_, i, 0, 0)),
        compiler_params=pltpu.CompilerParams(
            dimension_semantics=("parallel", "parallel"),
            vmem_limit_bytes=_VMEM_LIMIT),
    )(x, x, x, wk, b.reshape(1, cout))


def _conv_pair_kernel(xc_ref, xt_ref, xb_ref, w_ref, b_ref, o_ref, *,
                      th, w2, cin, cout, pool):
    i = pl.program_id(1)
    nh = pl.num_programs(1)
    x_c = xc_ref[0]
    x_t = xt_ref[0]
    x_b = xb_ref[0]
    zrow = jnp.zeros_like(x_t)
    x_t = jnp.where(i == 0, zrow, x_t)
    x_b = jnp.where(i == nh - 1, zrow, x_b)
    strip = jnp.concatenate([x_t, x_c, x_b], axis=0)

    zcol = jnp.zeros((th + 2, 1, cin), strip.dtype)
    t0 = jnp.concatenate([zcol, strip[:, :w2 - 1, cin:]], axis=1)
    t3 = jnp.concatenate([strip[:, 1:, :cin], zcol], axis=1)
    packed0 = jnp.concatenate([t0, strip], axis=2)
    packed1 = jnp.concatenate([strip, t3], axis=2)

    m = th * w2
    k3 = 3 * cin
    y0 = y1 = None
    for dy in range(3):
        d0 = jnp.dot(packed0[dy:dy + th].reshape(m, k3), w_ref[dy],
                     preferred_element_type=jnp.float32)
        d1 = jnp.dot(packed1[dy:dy + th].reshape(m, k3), w_ref[dy],
                     preferred_element_type=jnp.float32)
        y0 = d0 if y0 is None else y0 + d0
        y1 = d1 if y1 is None else y1 + d1
    y0 = jnp.maximum(y0 + b_ref[...], 0.0)
    y1 = jnp.maximum(y1 + b_ref[...], 0.0)

    if pool:
        y = jnp.maximum(y0, y1)
        y = y.reshape(th // 2, 2, w2, cout)
        y = jnp.max(y, axis=1)
        o_ref[...] = y.reshape(1, th // 2, w2, cout).astype(o_ref.dtype)
    else:
        y = jnp.concatenate([y0, y1], axis=1)
        o_ref[...] = y.reshape(1, th, w2, 2 * cout).astype(o_ref.dtype)


def _conv_pair(x, w, b, *, th, pool):
    n, h, w2, cin2 = x.shape
    cin = cin2 // 2
    cout = w.shape[-1]
    wk = w.reshape(3, 3 * cin, cout)
    ho = h // 2 if pool else h
    tho = th // 2 if pool else th
    co = cout if pool else 2 * cout
    kfn = functools.partial(_conv_pair_kernel, th=th, w2=w2, cin=cin,
                            cout=cout, pool=pool)
    return pl.pallas_call(
        kfn,
        out_shape=jax.ShapeDtypeStruct((n, ho, w2, co), x.dtype),
        grid=(n, h // th),
        in_specs=[
            pl.BlockSpec((1, th, w2, cin2), lambda n_, i: (n_, i, 0, 0)),
            pl.BlockSpec((1, 1, w2, cin2),
                         lambda n_, i: (n_, jnp.maximum(i * th - 1, 0), 0, 0)),
            pl.BlockSpec((1, 1, w2, cin2),
                         lambda n_, i: (n_, jnp.minimum(i * th + th, h - 1),
                                        0, 0)),
            pl.BlockSpec((3, 3 * cin, cout), lambda n_, i: (0, 0, 0)),
            pl.BlockSpec((1, cout), lambda n_, i: (0, 0)),
        ],
        out_specs=pl.BlockSpec((1, tho, w2, co), lambda n_, i: (n_, i, 0, 0)),
        compiler_params=pltpu.CompilerParams(
            dimension_semantics=("parallel", "parallel"),
            vmem_limit_bytes=_VMEM_LIMIT),
    )(x, x, x, wk, b.reshape(1, cout))


def _stem_kernel(x_ref, w_ref, b_ref, o_ref):
    ys = []
    for j in range(4):
        d = jnp.dot(x_ref[:, 27 * j:27 * (j + 1)], w_ref[...],
                    preferred_element_type=jnp.float32)
        ys.append(jnp.maximum(d + b_ref[...], 0.0))
    o_ref[...] = jnp.concatenate(ys, axis=1).astype(o_ref.dtype)


def _stem(x, w, b):
    n, h, ww, cin = x.shape
    cout = w.shape[-1]
    xp = jnp.pad(x, ((0, 0), (1, 1), (1, 1), (0, 0)))
    taps = [xp[:, dy:dy + h, dx:dx + ww, :]
            for dy in range(3) for dx in range(3)]
    m4 = n * h * ww // 4
    xi = jnp.concatenate(taps, axis=-1).reshape(m4, 4 * 9 * cin)
    wk = w.reshape(9 * cin, cout)
    tm = min(2048, m4)
    out = pl.pallas_call(
        _stem_kernel,
        out_shape=jax.ShapeDtypeStruct((m4, 4 * cout), x.dtype),
        grid=(m4 // tm,),
        in_specs=[
            pl.BlockSpec((tm, 4 * 9 * cin), lambda i: (i, 0)),
            pl.BlockSpec((9 * cin, cout), lambda i: (0, 0)),
            pl.BlockSpec((1, cout), lambda i: (0, 0)),
        ],
        out_specs=pl.BlockSpec((tm, 4 * cout), lambda i: (i, 0)),
        compiler_params=pltpu.CompilerParams(
            dimension_semantics=("parallel",),
            vmem_limit_bytes=_VMEM_LIMIT),
    )(xi, wk, b.reshape(1, cout))
    return out.reshape(n, h, ww // 4, 4 * cout)


def _conv_quad_kernel(xc_ref, xt_ref, xb_ref, w_ref, b_ref, o_ref, *,
                      th, w4, cin, cout):
    i = pl.program_id(1)
    nh = pl.num_programs(1)
    x_c = xc_ref[0]
    x_t = xt_ref[0]
    x_b = xb_ref[0]
    zrow = jnp.zeros_like(x_t)
    x_t = jnp.where(i == 0, zrow, x_t)
    x_b = jnp.where(i == nh - 1, zrow, x_b)
    strip = jnp.concatenate([x_t, x_c, x_b], axis=0)

    zcol = jnp.zeros((th + 2, 1, cin), strip.dtype)
    t_prev = jnp.concatenate([zcol, strip[:, :w4 - 1, 3 * cin:]], axis=1)
    t_next = jnp.concatenate([strip[:, 1:, :cin], zcol], axis=1)
    ops = [
        jnp.concatenate([t_prev, strip[:, :, :2 * cin]], axis=2),
        strip[:, :, :3 * cin],
        strip[:, :, cin:],
        jnp.concatenate([strip[:, :, 2 * cin:], t_next], axis=2),
    ]

    m = th * w4
    k3 = 3 * cin
    ys = [None] * 4
    for dy in range(3):
        for j in range(4):
            d = jnp.dot(ops[j][dy:dy + th].reshape(m, k3), w_ref[dy],
                        preferred_element_type=jnp.float32)
            ys[j] = d if ys[j] is None else ys[j] + d
    ys = [jnp.maximum(yj + b_ref[...], 0.0) for yj in ys]

    z = jnp.concatenate([jnp.maximum(ys[0], ys[1]),
                         jnp.maximum(ys[2], ys[3])], axis=1)
    z = z.reshape(th // 2, 2, w4, 2 * cout)
    z = jnp.max(z, axis=1)
    o_ref[...] = z.reshape(1, th // 2, w4, 2 * cout).astype(o_ref.dtype)


def _conv_quad(x, w, b, *, th):
    n, h, w4, cin4 = x.shape
    cin = cin4 // 4
    cout = w.shape[-1]
    wk = w.reshape(3, 3 * cin, cout)
    kfn = functools.partial(_conv_quad_kernel, th=th, w4=w4, cin=cin,
                            cout=cout)
    return pl.pallas_call(
        kfn,
        out_shape=jax.ShapeDtypeStruct((n, h // 2, w4, 2 * cout), x.dtype),
        grid=(n, h // th),
        in_specs=[
            pl.BlockSpec((1, th, w4, cin4), lambda n_, i: (n_, i, 0, 0)),
            pl.BlockSpec((1, 1, w4, cin4),
                         lambda n_, i: (n_, jnp.maximum(i * th - 1, 0), 0, 0)),
            pl.BlockSpec((1, 1, w4, cin4),
                         lambda n_, i: (n_, jnp.minimum(i * th + th, h - 1),
                                        0, 0)),
            pl.BlockSpec((3, 3 * cin, cout), lambda n_, i: (0, 0, 0)),
            pl.BlockSpec((1, cout), lambda n_, i: (0, 0)),
        ],
        out_specs=pl.BlockSpec((1, th // 2, w4, 2 * cout),
                               lambda n_, i: (n_, i, 0, 0)),
        compiler_params=pltpu.CompilerParams(
            dimension_semantics=("parallel", "parallel"),
            vmem_limit_bytes=_VMEM_LIMIT),
    )(x, x, x, wk, b.reshape(1, cout))


def _fc_int8_kernel(x_ref, wq_ref, s_ref, b_ref, o_ref, acc_ref, *, relu):
    k = pl.program_id(1)

    @pl.when(k == 0)
    def _():
        acc_ref[...] = jnp.zeros_like(acc_ref)

    w = wq_ref[...].astype(jnp.bfloat16)
    acc_ref[...] += jnp.dot(x_ref[...], w, preferred_element_type=jnp.float32)

    @pl.when(k == pl.num_programs(1) - 1)
    def _():
        y = acc_ref[...] * s_ref[...] + b_ref[...]
        if relu:
            y = jnp.maximum(y, 0.0)
        o_ref[...] = y.astype(o_ref.dtype)


def _fc_int8(x, wq, s, b, *, relu, tk, tn):
    bsz, kdim = x.shape
    ndim = wq.shape[1]
    kfn = functools.partial(_fc_int8_kernel, relu=relu)
    return pl.pallas_call(
        kfn,
        out_shape=jax.ShapeDtypeStruct((bsz, ndim), x.dtype),
        grid_spec=pltpu.PrefetchScalarGridSpec(
            num_scalar_prefetch=0,
            grid=(ndim // tn, kdim // tk),
            in_specs=[
                pl.BlockSpec((bsz, tk), lambda j, k: (0, k)),
                pl.BlockSpec((tk, tn), lambda j, k: (k, j)),
                pl.BlockSpec((1, tn), lambda j, k: (0, j)),
                pl.BlockSpec((1, tn), lambda j, k: (0, j)),
            ],
            out_specs=pl.BlockSpec((bsz, tn), lambda j, k: (0, j)),
            scratch_shapes=[pltpu.VMEM((bsz, tn), jnp.float32)],
        ),
        compiler_params=pltpu.CompilerParams(
            dimension_semantics=("parallel", "arbitrary"),
            vmem_limit_bytes=_VMEM_LIMIT),
    )(x, wq, s.reshape(1, ndim), b.reshape(1, ndim))


def _fc_kernel(x_ref, w_ref, b_ref, o_ref):
    y = jnp.dot(x_ref[...], w_ref[...], preferred_element_type=jnp.float32)
    o_ref[...] = y + b_ref[...]


def _fc_small(x, w, b):
    bsz = x.shape[0]
    ndim = w.shape[1]
    return pl.pallas_call(
        _fc_kernel,
        out_shape=jax.ShapeDtypeStruct((bsz, ndim), jnp.float32),
        compiler_params=pltpu.CompilerParams(
            vmem_limit_bytes=_VMEM_LIMIT),
    )(x, w, b.reshape(1, ndim))


_PLAN = [
    (4, 28, False, 56, 56),
    (5, 28, False, 56, 56),
    (6, 28, True, 56, 28),
    (7, 28, False, 28, 28),
    (8, 28, False, 28, 28),
    (9, 28, True, 28, 14),
    (10, 14, False, 14, 14),
    (11, 14, False, 14, 14),
    (12, 14, True, 14, 7),
]


def kernel(x,
           conv0_w, conv0_b, conv1_w, conv1_b, conv2_w, conv2_b,
           conv3_w, conv3_b, conv4_w, conv4_b, conv5_w, conv5_b,
           conv6_w, conv6_b, conv7_w, conv7_b, conv8_w, conv8_b,
           conv9_w, conv9_b, conv10_w, conv10_b, conv11_w, conv11_b,
           conv12_w, conv12_b,
           fc0_wq, fc0_s, fc0_b, fc1_wq, fc1_s, fc1_b, fc2_w, fc2_b):
    convs = [(conv0_w, conv0_b), (conv1_w, conv1_b), (conv2_w, conv2_b),
             (conv3_w, conv3_b), (conv4_w, conv4_b), (conv5_w, conv5_b),
             (conv6_w, conv6_b), (conv7_w, conv7_b), (conv8_w, conv8_b),
             (conv9_w, conv9_b), (conv10_w, conv10_b), (conv11_w, conv11_b),
             (conv12_w, conv12_b)]

    h = jnp.transpose(x, (0, 2, 3, 1)).astype(jnp.bfloat16)
    h = _stem(h, convs[0][0], convs[0][1])
    h = _conv_quad(h, convs[1][0], convs[1][1], th=28)
    h = _conv_pair(h, convs[2][0], convs[2][1], th=28, pool=False)
    h = _conv_pair(h, convs[3][0], convs[3][1], th=28, pool=True)
    for li, th, pool, wt, wpo in _PLAN:
        w, b = convs[li]
        h = _conv(h, w, b, th=th, pool=pool, wt=wt, wpo=wpo)

    h = h[:, :, :7, :]
    h = jnp.transpose(h, (0, 3, 1, 2)).reshape(h.shape[0], -1)
    h = _fc_int8(h, fc0_wq, fc0_s, fc0_b, relu=True, tk=1792, tn=2048)
    h = _fc_int8(h, fc1_wq, fc1_s, fc1_b, relu=True, tk=2048, tn=2048)
    return _fc_small(h, fc2_w, fc2_b)
